# baseline (speedup 1.0000x reference)
import functools

import jax
import jax.numpy as jnp
from jax import lax
from jax.experimental import pallas as pl
from jax.experimental.pallas import tpu as pltpu
from jax.experimental.pallas import tpu_sc as plsc

B = 8
HW = 1024
D = 512
N = 512
K = 48
KTOP = 24
NC, NS = 2, 16
NW = NC * NS
NPT = N // NW
AW = HW // 4


def _gram_body(feat_ref, pid_ref, g_ref, p_ref):
    @pl.when(pl.program_id(0) == 0)
    def _():
        iota = lax.broadcasted_iota(jnp.int32, (N, HW), 1)
        p_ref[...] = (iota == pid_ref[...]).astype(jnp.float32)

    x = feat_ref[0]
    nsq = jnp.sum(x * x, axis=1, keepdims=True)
    nrm = jnp.maximum(jnp.sqrt(nsq), 1e-12)
    xn = x / nrm
    c = lax.dot_general(p_ref[...], xn, (((1,), (0,)), ((), ())),
                        preferred_element_type=jnp.float32)
    g_ref[...] = lax.dot_general(c, xn, (((1,), (1,)), ((), ())),
                                 preferred_element_type=jnp.float32)


_gram = pl.pallas_call(
    _gram_body,
    grid=(B,),
    in_specs=[pl.BlockSpec((1, HW, D), lambda i: (i, 0, 0)),
              pl.BlockSpec((N, 1), lambda i: (0, 0))],
    out_specs=pl.BlockSpec((N, HW), lambda i: (i, 0)),
    out_shape=jax.ShapeDtypeStruct((B * N, HW), jnp.float32),
    scratch_shapes=[pltpu.VMEM((N, HW), jnp.float32)],
)


def _sc_body(g_hbm, pid_hbm, lid_hbm, top_hbm, a_hbm,
             lidblk, pidblk, grows, topblk, ablk, sem):
    wid = lax.axis_index("s") * NC + lax.axis_index("c")
    nbase = wid * NPT
    pltpu.sync_copy(lid_hbm.at[pl.ds(nbase * K, NPT * K)], lidblk)
    pltpu.sync_copy(pid_hbm.at[pl.ds(nbase, NPT)], pidblk)
    lanes = lax.iota(jnp.int32, 16)
    lane0 = lanes == 0
    one = jnp.full((16,), 1, jnp.int32)
    eight = jnp.full((16,), 8, jnp.int32)
    NEG = jnp.float32(-3.0)

    def b_body(b, carry):
        task0 = b * N + nbase
        pltpu.async_copy(g_hbm.at[pl.ds(task0, NPT)], grows, sem).wait()

        def process(nidx):
            nsp = jnp.broadcast_to(nidx, (16,)).astype(jnp.int32)
            v0 = plsc.load_gather(
                grows, [nsp, plsc.load_gather(lidblk, [nsp * K + lanes])])
            v1 = plsc.load_gather(
                grows, [nsp, plsc.load_gather(lidblk, [nsp * K + lanes + 16])])
            v2 = plsc.load_gather(
                grows, [nsp, plsc.load_gather(lidblk, [nsp * K + lanes + 32])])
            zz = jnp.zeros((16,), jnp.int32)
            for i in range(AW // 16):
                plsc.store_scatter(ablk, [nsp, lanes + i * 16], zz)
            pidn = plsc.load_gather(pidblk, [nsp])
            plsc.addupdate_scatter(
                ablk, [nsp, pidn & 255],
                one << ((pidn >> 8) << 3), mask=lane0)
            for t in range(KTOP):
                m = jnp.maximum(jnp.maximum(v0, v1), v2)
                for s in (8, 4, 2, 1):
                    sh = lax.gather(
                        m, (lanes ^ s).reshape(16, 1),
                        lax.GatherDimensionNumbers(
                            offset_dims=(), collapsed_slice_dims=(0,),
                            start_index_map=(0,)),
                        slice_sizes=(1,),
                        mode=lax.GatherScatterMode.PROMISE_IN_BOUNDS)
                    m = jnp.maximum(m, sh)
                f0 = jnp.broadcast_to(plsc.all_reduce_ffs(v0 == m), (16,))
                f1 = jnp.broadcast_to(plsc.all_reduce_ffs(v1 == m), (16,))
                f2 = jnp.broadcast_to(plsc.all_reduce_ffs(v2 == m), (16,))
                kk = jnp.where(f0 < 16, f0,
                               jnp.where(f1 < 16, f1 + 16, f2 + 32))
                kk = kk.astype(jnp.int32)
                plsc.store_scatter(
                    topblk, [nsp * KTOP + jnp.full((16,), t, jnp.int32)], kk,
                    mask=lane0)
                lidk = plsc.load_gather(lidblk, [nsp * K + kk])
                plsc.addupdate_scatter(
                    ablk, [nsp, lidk & 255],
                    one << ((lidk >> 8) << 3), mask=lane0)
                v0 = jnp.where(kk == lanes, NEG, v0)
                v1 = jnp.where(kk == lanes + 16, NEG, v1)
                v2 = jnp.where(kk == lanes + 32, NEG, v2)

        def n_body(n, c2):
            process(n)
            process(n + NPT // 2)
            return c2

        lax.fori_loop(0, NPT // 2, n_body, 0)
        pltpu.sync_copy(topblk, top_hbm.at[pl.ds(task0 * KTOP, NPT * KTOP)])
        pltpu.sync_copy(ablk, a_hbm.at[pl.ds(task0, NPT)])
        return carry

    lax.fori_loop(0, B, b_body, 0)


_sc_topk = functools.partial(
    pl.kernel,
    out_type=[jax.ShapeDtypeStruct((B * N * KTOP,), jnp.int32),
              jax.ShapeDtypeStruct((B * N, AW), jnp.int32)],
    mesh=plsc.VectorSubcoreMesh(core_axis_name="c", subcore_axis_name="s"),
    compiler_params=pltpu.CompilerParams(use_tc_tiling_on_sc=False,
                                         needs_layout_passes=False),
    scratch_types=[
        pltpu.VMEM((NPT * K,), jnp.int32),
        pltpu.VMEM((NPT,), jnp.int32),
        pltpu.VMEM((NPT, HW), jnp.float32),
        pltpu.VMEM((NPT * KTOP,), jnp.int32),
        pltpu.VMEM((NPT, AW), jnp.int32),
        pltpu.SemaphoreType.DMA,
    ],
)(_sc_body)


def _mlp_body(a_ref, feat_ref, w1_ref, b1_ref, g1_ref, be1_ref,
              w2_ref, b2_ref, out_ref):
    aw = a_ref[...]
    parts = [(aw >> (8 * t)) & 255 for t in range(4)]
    cnt = jnp.concatenate(parts, axis=1)
    a = cnt.astype(jnp.float32) * (1.0 / 25.0)
    f = feat_ref[0]
    xs = lax.dot_general(a, f, (((1,), (0,)), ((), ())),
                         preferred_element_type=jnp.float32)
    h = lax.dot_general(xs, w1_ref[...], (((1,), (0,)), ((), ())),
                        preferred_element_type=jnp.float32) + b1_ref[...]
    h = (h / jnp.sqrt(jnp.float32(1.0 + 1e-5))) * g1_ref[...] + be1_ref[...]
    h = jnp.maximum(h, 0.0)
    out_ref[...] = lax.dot_general(h, w2_ref[...], (((1,), (0,)), ((), ())),
                                   preferred_element_type=jnp.float32) + b2_ref[...]


_mlp = pl.pallas_call(
    _mlp_body,
    grid=(B,),
    in_specs=[
        pl.BlockSpec((N, AW), lambda i: (i, 0)),
        pl.BlockSpec((1, HW, D), lambda i: (i, 0, 0)),
        pl.BlockSpec((D, 2 * D), lambda i: (0, 0)),
        pl.BlockSpec((1, 2 * D), lambda i: (0, 0)),
        pl.BlockSpec((1, 2 * D), lambda i: (0, 0)),
        pl.BlockSpec((1, 2 * D), lambda i: (0, 0)),
        pl.BlockSpec((2 * D, D // 2), lambda i: (0, 0)),
        pl.BlockSpec((1, D // 2), lambda i: (0, 0)),
    ],
    out_specs=pl.BlockSpec((N, D // 2), lambda i: (i, 0)),
    out_shape=jax.ShapeDtypeStruct((B * N, D // 2), jnp.float32),
)


def kernel(patch_size, feats, num_patches, patch_ids, patch_local_ids,
           W1, b1, gamma1, beta1, W2, b2):
    feat = feats[0]
    feat_r = jnp.transpose(feat, (0, 2, 3, 1)).reshape(B, HW, D)
    pid = (patch_ids[0][:, 0] if patch_ids.ndim == 3
           else patch_ids[0]).astype(jnp.int32)
    lid = patch_local_ids[0].astype(jnp.int32)

    G = _gram(feat_r, pid.reshape(N, 1))
    top_idx, A = _sc_topk(G, pid, lid.reshape(-1))
    out = _mlp(A, feat_r, W1,
               b1.reshape(1, -1), gamma1.reshape(1, -1), beta1.reshape(1, -1),
               W2, b2.reshape(1, -1))
    return (out, pid, lid, top_idx.reshape(B * N, KTOP, 1))


# speedup vs baseline: 1.0391x; 1.0391x over previous
import functools

import jax
import jax.numpy as jnp
from jax import lax
from jax.experimental import pallas as pl
from jax.experimental.pallas import tpu as pltpu
from jax.experimental.pallas import tpu_sc as plsc

B = 8
HW = 1024
D = 512
N = 512
K = 48
KTOP = 24
NC, NS = 2, 16
NW = NC * NS
NPT = N // NW
AW = HW // 4


def _gram_body(feat_ref, pid_ref, g_ref, p_ref):
    @pl.when(pl.program_id(0) == 0)
    def _():
        iota = lax.broadcasted_iota(jnp.int32, (N, HW), 1)
        p_ref[...] = (iota == pid_ref[...]).astype(jnp.float32)

    x = feat_ref[0]
    nsq = jnp.sum(x * x, axis=1, keepdims=True)
    nrm = jnp.maximum(jnp.sqrt(nsq), 1e-12)
    xn = x / nrm
    c = lax.dot_general(p_ref[...], xn, (((1,), (0,)), ((), ())),
                        preferred_element_type=jnp.float32)
    g_ref[...] = lax.dot_general(c, xn, (((1,), (1,)), ((), ())),
                                 preferred_element_type=jnp.float32)


_gram = pl.pallas_call(
    _gram_body,
    grid=(B,),
    in_specs=[pl.BlockSpec((1, HW, D), lambda i: (i, 0, 0)),
              pl.BlockSpec((N, 1), lambda i: (0, 0))],
    out_specs=pl.BlockSpec((N, HW), lambda i: (i, 0)),
    out_shape=jax.ShapeDtypeStruct((B * N, HW), jnp.float32),
    scratch_shapes=[pltpu.VMEM((N, HW), jnp.float32)],
)


def _sc_body(g_hbm, pid_hbm, lid_hbm, top_hbm, a_hbm,
             lidblk, pidblk, grows, topblk, ablk, sem):
    wid = lax.axis_index("s") * NC + lax.axis_index("c")
    nbase = wid * NPT
    pltpu.sync_copy(lid_hbm.at[pl.ds(nbase * K, NPT * K)], lidblk)
    pltpu.sync_copy(pid_hbm.at[pl.ds(nbase, NPT)], pidblk)
    lanes = lax.iota(jnp.int32, 16)
    lane0 = lanes == 0
    one = jnp.full((16,), 1, jnp.int32)
    eight = jnp.full((16,), 8, jnp.int32)
    NEG = jnp.float32(-3.0)

    def b_body(b, carry):
        task0 = b * N + nbase
        pltpu.async_copy(g_hbm.at[pl.ds(task0, NPT)], grows, sem).wait()

        def process(nidx):
            nsp = jnp.broadcast_to(nidx, (16,)).astype(jnp.int32)
            v0 = plsc.load_gather(
                grows, [nsp, plsc.load_gather(lidblk, [nsp * K + lanes])])
            v1 = plsc.load_gather(
                grows, [nsp, plsc.load_gather(lidblk, [nsp * K + lanes + 16])])
            v2 = plsc.load_gather(
                grows, [nsp, plsc.load_gather(lidblk, [nsp * K + lanes + 32])])
            zz = jnp.zeros((16,), jnp.int32)
            for i in range(AW // 16):
                plsc.store_scatter(ablk, [nsp, lanes + i * 16], zz)
            pidn = plsc.load_gather(pidblk, [nsp])
            plsc.addupdate_scatter(
                ablk, [nsp, pidn & 255],
                one << ((pidn >> 8) << 3), mask=lane0)
            for t in range(KTOP):
                vm = jnp.maximum(jnp.maximum(v0, v1), v2)
                m = jnp.broadcast_to(jnp.max(vm), (16,))
                f0 = jnp.broadcast_to(plsc.all_reduce_ffs(v0 == m), (16,))
                f1 = jnp.broadcast_to(plsc.all_reduce_ffs(v1 == m), (16,))
                f2 = jnp.broadcast_to(plsc.all_reduce_ffs(v2 == m), (16,))
                kk = jnp.where(f0 < 16, f0,
                               jnp.where(f1 < 16, f1 + 16, f2 + 32))
                kk = kk.astype(jnp.int32)
                plsc.store_scatter(
                    topblk, [nsp * KTOP + jnp.full((16,), t, jnp.int32)], kk,
                    mask=lane0)
                lidk = plsc.load_gather(lidblk, [nsp * K + kk])
                plsc.addupdate_scatter(
                    ablk, [nsp, lidk & 255],
                    one << ((lidk >> 8) << 3), mask=lane0)
                v0 = jnp.where(kk == lanes, NEG, v0)
                v1 = jnp.where(kk == lanes + 16, NEG, v1)
                v2 = jnp.where(kk == lanes + 32, NEG, v2)

        def n_body(n, c2):
            process(n)
            process(n + NPT // 2)
            return c2

        lax.fori_loop(0, NPT // 2, n_body, 0)
        pltpu.sync_copy(topblk, top_hbm.at[pl.ds(task0 * KTOP, NPT * KTOP)])
        pltpu.sync_copy(ablk, a_hbm.at[pl.ds(task0, NPT)])
        return carry

    lax.fori_loop(0, B, b_body, 0)


_sc_topk = functools.partial(
    pl.kernel,
    out_type=[jax.ShapeDtypeStruct((B * N * KTOP,), jnp.int32),
              jax.ShapeDtypeStruct((B * N, AW), jnp.int32)],
    mesh=plsc.VectorSubcoreMesh(core_axis_name="c", subcore_axis_name="s"),
    compiler_params=pltpu.CompilerParams(use_tc_tiling_on_sc=False,
                                         needs_layout_passes=False),
    scratch_types=[
        pltpu.VMEM((NPT * K,), jnp.int32),
        pltpu.VMEM((NPT,), jnp.int32),
        pltpu.VMEM((NPT, HW), jnp.float32),
        pltpu.VMEM((NPT * KTOP,), jnp.int32),
        pltpu.VMEM((NPT, AW), jnp.int32),
        pltpu.SemaphoreType.DMA,
    ],
)(_sc_body)


def _mlp_body(a_ref, feat_ref, w1_ref, b1_ref, g1_ref, be1_ref,
              w2_ref, b2_ref, out_ref):
    aw = a_ref[...]
    parts = [(aw >> (8 * t)) & 255 for t in range(4)]
    cnt = jnp.concatenate(parts, axis=1)
    a = cnt.astype(jnp.float32) * (1.0 / 25.0)
    f = feat_ref[0]
    xs = lax.dot_general(a, f, (((1,), (0,)), ((), ())),
                         preferred_element_type=jnp.float32)
    h = lax.dot_general(xs, w1_ref[...], (((1,), (0,)), ((), ())),
                        preferred_element_type=jnp.float32) + b1_ref[...]
    h = (h / jnp.sqrt(jnp.float32(1.0 + 1e-5))) * g1_ref[...] + be1_ref[...]
    h = jnp.maximum(h, 0.0)
    out_ref[...] = lax.dot_general(h, w2_ref[...], (((1,), (0,)), ((), ())),
                                   preferred_element_type=jnp.float32) + b2_ref[...]


_mlp = pl.pallas_call(
    _mlp_body,
    grid=(B,),
    in_specs=[
        pl.BlockSpec((N, AW), lambda i: (i, 0)),
        pl.BlockSpec((1, HW, D), lambda i: (i, 0, 0)),
        pl.BlockSpec((D, 2 * D), lambda i: (0, 0)),
        pl.BlockSpec((1, 2 * D), lambda i: (0, 0)),
        pl.BlockSpec((1, 2 * D), lambda i: (0, 0)),
        pl.BlockSpec((1, 2 * D), lambda i: (0, 0)),
        pl.BlockSpec((2 * D, D // 2), lambda i: (0, 0)),
        pl.BlockSpec((1, D // 2), lambda i: (0, 0)),
    ],
    out_specs=pl.BlockSpec((N, D // 2), lambda i: (i, 0)),
    out_shape=jax.ShapeDtypeStruct((B * N, D // 2), jnp.float32),
)


def kernel(patch_size, feats, num_patches, patch_ids, patch_local_ids,
           W1, b1, gamma1, beta1, W2, b2):
    feat = feats[0]
    feat_r = jnp.transpose(feat, (0, 2, 3, 1)).reshape(B, HW, D)
    pid = (patch_ids[0][:, 0] if patch_ids.ndim == 3
           else patch_ids[0]).astype(jnp.int32)
    lid = patch_local_ids[0].astype(jnp.int32)

    G = _gram(feat_r, pid.reshape(N, 1))
    top_idx, A = _sc_topk(G, pid, lid.reshape(-1))
    out = _mlp(A, feat_r, W1,
               b1.reshape(1, -1), gamma1.reshape(1, -1), beta1.reshape(1, -1),
               W2, b2.reshape(1, -1))
    return (out, pid, lid, top_idx.reshape(B * N, KTOP, 1))


# speedup vs baseline: 1.0491x; 1.0097x over previous
import functools

import jax
import jax.numpy as jnp
from jax import lax
from jax.experimental import pallas as pl
from jax.experimental.pallas import tpu as pltpu
from jax.experimental.pallas import tpu_sc as plsc

B = 8
HW = 1024
D = 512
N = 512
K = 48
KTOP = 24
NC, NS = 2, 16
NW = NC * NS
NPT = N // NW
AW = HW // 4


def _gram_body(feat_ref, pid_ref, g_ref, p_ref):
    @pl.when(pl.program_id(0) == 0)
    def _():
        iota = lax.broadcasted_iota(jnp.int32, (N, HW), 1)
        p_ref[...] = (iota == pid_ref[...]).astype(jnp.float32)

    x = feat_ref[0]
    nsq = jnp.sum(x * x, axis=1, keepdims=True)
    nrm = jnp.maximum(jnp.sqrt(nsq), 1e-12)
    xn = x / nrm
    c = lax.dot_general(p_ref[...], xn, (((1,), (0,)), ((), ())),
                        preferred_element_type=jnp.float32)
    g_ref[...] = lax.dot_general(c, xn, (((1,), (1,)), ((), ())),
                                 preferred_element_type=jnp.float32)


_gram = pl.pallas_call(
    _gram_body,
    grid=(B,),
    in_specs=[pl.BlockSpec((1, HW, D), lambda i: (i, 0, 0)),
              pl.BlockSpec((N, 1), lambda i: (0, 0))],
    out_specs=pl.BlockSpec((N, HW), lambda i: (i, 0)),
    out_shape=jax.ShapeDtypeStruct((B * N, HW), jnp.float32),
    scratch_shapes=[pltpu.VMEM((N, HW), jnp.float32)],
)


def _sc_body(g_hbm, pid_hbm, lid_hbm, top_hbm, a_hbm,
             lidblk, pidblk, grows, topblk, ablk, sem):
    wid = lax.axis_index("s") * NC + lax.axis_index("c")
    nbase = wid * NPT
    pltpu.sync_copy(lid_hbm.at[pl.ds(nbase * K, NPT * K)], lidblk)
    pltpu.sync_copy(pid_hbm.at[pl.ds(nbase, NPT)], pidblk)
    lanes = lax.iota(jnp.int32, 16)
    lane0 = lanes == 0
    one = jnp.full((16,), 1, jnp.int32)
    eight = jnp.full((16,), 8, jnp.int32)
    NEG = jnp.float32(-3.0)

    def b_body(b, carry):
        task0 = b * N + nbase
        pltpu.async_copy(g_hbm.at[pl.ds(task0, NPT)], grows, sem).wait()

        def n_body(n, c2):
            nsp = jnp.broadcast_to(n, (16,)).astype(jnp.int32)
            v0 = plsc.load_gather(
                grows, [nsp, plsc.load_gather(lidblk, [nsp * K + lanes])])
            v1 = plsc.load_gather(
                grows, [nsp, plsc.load_gather(lidblk, [nsp * K + lanes + 16])])
            v2 = plsc.load_gather(
                grows, [nsp, plsc.load_gather(lidblk, [nsp * K + lanes + 32])])
            zz = jnp.zeros((16,), jnp.int32)
            for i in range(AW // 16):
                plsc.store_scatter(ablk, [nsp, lanes + i * 16], zz)
            pidn = plsc.load_gather(pidblk, [nsp])
            plsc.addupdate_scatter(
                ablk, [nsp, pidn & 255],
                one << ((pidn >> 8) << 3), mask=lane0)
            for t in range(KTOP):
                vm = jnp.maximum(jnp.maximum(v0, v1), v2)
                m = jnp.broadcast_to(jnp.max(vm), (16,))
                f0 = jnp.broadcast_to(plsc.all_reduce_ffs(v0 == m), (16,))
                f1 = jnp.broadcast_to(plsc.all_reduce_ffs(v1 == m), (16,))
                f2 = jnp.broadcast_to(plsc.all_reduce_ffs(v2 == m), (16,))
                kk = jnp.where(f0 < 16, f0,
                               jnp.where(f1 < 16, f1 + 16, f2 + 32))
                kk = kk.astype(jnp.int32)
                plsc.store_scatter(
                    topblk, [nsp * KTOP + jnp.full((16,), t, jnp.int32)], kk,
                    mask=lane0)
                lidk = plsc.load_gather(lidblk, [nsp * K + kk])
                plsc.addupdate_scatter(
                    ablk, [nsp, lidk & 255],
                    one << ((lidk >> 8) << 3), mask=lane0)
                v0 = jnp.where(kk == lanes, NEG, v0)
                v1 = jnp.where(kk == lanes + 16, NEG, v1)
                v2 = jnp.where(kk == lanes + 32, NEG, v2)
            return c2

        lax.fori_loop(0, NPT, n_body, 0)
        pltpu.sync_copy(topblk, top_hbm.at[pl.ds(task0 * KTOP, NPT * KTOP)])
        pltpu.sync_copy(ablk, a_hbm.at[pl.ds(task0, NPT)])
        return carry

    lax.fori_loop(0, B, b_body, 0)


_sc_topk = functools.partial(
    pl.kernel,
    out_type=[jax.ShapeDtypeStruct((B * N * KTOP,), jnp.int32),
              jax.ShapeDtypeStruct((B * N, AW), jnp.int32)],
    mesh=plsc.VectorSubcoreMesh(core_axis_name="c", subcore_axis_name="s"),
    compiler_params=pltpu.CompilerParams(use_tc_tiling_on_sc=False,
                                         needs_layout_passes=False),
    scratch_types=[
        pltpu.VMEM((NPT * K,), jnp.int32),
        pltpu.VMEM((NPT,), jnp.int32),
        pltpu.VMEM((NPT, HW), jnp.float32),
        pltpu.VMEM((NPT * KTOP,), jnp.int32),
        pltpu.VMEM((NPT, AW), jnp.int32),
        pltpu.SemaphoreType.DMA,
    ],
)(_sc_body)


def _mlp_body(a_ref, feat_ref, w1_ref, b1_ref, g1_ref, be1_ref,
              w2_ref, b2_ref, out_ref):
    aw = a_ref[...]
    parts = [(aw >> (8 * t)) & 255 for t in range(4)]
    cnt = jnp.concatenate(parts, axis=1)
    a = cnt.astype(jnp.float32) * (1.0 / 25.0)
    f = feat_ref[0]
    xs = lax.dot_general(a, f, (((1,), (0,)), ((), ())),
                         preferred_element_type=jnp.float32)
    h = lax.dot_general(xs, w1_ref[...], (((1,), (0,)), ((), ())),
                        preferred_element_type=jnp.float32) + b1_ref[...]
    h = (h / jnp.sqrt(jnp.float32(1.0 + 1e-5))) * g1_ref[...] + be1_ref[...]
    h = jnp.maximum(h, 0.0)
    out_ref[...] = lax.dot_general(h, w2_ref[...], (((1,), (0,)), ((), ())),
                                   preferred_element_type=jnp.float32) + b2_ref[...]


_mlp = pl.pallas_call(
    _mlp_body,
    grid=(B,),
    in_specs=[
        pl.BlockSpec((N, AW), lambda i: (i, 0)),
        pl.BlockSpec((1, HW, D), lambda i: (i, 0, 0)),
        pl.BlockSpec((D, 2 * D), lambda i: (0, 0)),
        pl.BlockSpec((1, 2 * D), lambda i: (0, 0)),
        pl.BlockSpec((1, 2 * D), lambda i: (0, 0)),
        pl.BlockSpec((1, 2 * D), lambda i: (0, 0)),
        pl.BlockSpec((2 * D, D // 2), lambda i: (0, 0)),
        pl.BlockSpec((1, D // 2), lambda i: (0, 0)),
    ],
    out_specs=pl.BlockSpec((N, D // 2), lambda i: (i, 0)),
    out_shape=jax.ShapeDtypeStruct((B * N, D // 2), jnp.float32),
)


def kernel(patch_size, feats, num_patches, patch_ids, patch_local_ids,
           W1, b1, gamma1, beta1, W2, b2):
    feat = feats[0]
    feat_r = jnp.transpose(feat, (0, 2, 3, 1)).reshape(B, HW, D)
    pid = (patch_ids[0][:, 0] if patch_ids.ndim == 3
           else patch_ids[0]).astype(jnp.int32)
    lid = patch_local_ids[0].astype(jnp.int32)

    G = _gram(feat_r, pid.reshape(N, 1))
    top_idx, A = _sc_topk(G, pid, lid.reshape(-1))
    out = _mlp(A, feat_r, W1,
               b1.reshape(1, -1), gamma1.reshape(1, -1), beta1.reshape(1, -1),
               W2, b2.reshape(1, -1))
    return (out, pid, lid, top_idx.reshape(B * N, KTOP, 1))
